# baseline (speedup 1.0000x reference)
import functools

import jax
import jax.numpy as jnp
import numpy as np
from jax.experimental import pallas as pl
from jax.experimental.pallas import tpu as pltpu

_N = 1000000
_B = 16
_BLK = 16384
_NBLK = (_N + _BLK - 1) // _BLK
_C = float(0.5 * np.log(2.0 * np.pi))
_NEG = -1e30


def _softplus(x):
    return jnp.maximum(x, 0.0) + jnp.log1p(jnp.exp(-jnp.abs(x)))


def _tc_loss_body(res_ref, nz_ref, t_ref, s_ref, e_ref, sig_ref,
                  w1_ref, b1_ref, w2_ref, b2_ref, w3_ref, b3_ref, w4_ref, b4_ref,
                  out_ref, ns_ref, m_ref, s_acc_ref, mw_ref, sw_ref):
    i = pl.program_id(0)

    @pl.when(i == 0)
    def _init():
        t = t_ref[...]
        s = s_ref[...]
        h = jnp.maximum(t * w1_ref[0:1, :] + s * w1_ref[1:2, :] + b1_ref[0:1, :], 0.0)
        acc2 = jnp.zeros((_B, 32), jnp.float32) + b2_ref[0:1, :]
        for k in range(32):
            acc2 = acc2 + h[:, k:k + 1] * w2_ref[k:k + 1, :]
        h2 = jnp.maximum(acc2, 0.0)
        acc3 = jnp.zeros((_B, 16), jnp.float32) + b3_ref[0:1, :]
        for k in range(32):
            acc3 = acc3 + h2[:, k:k + 1] * w3_ref[k:k + 1, :]
        h3 = jnp.maximum(acc3, 0.0)
        acc4 = jnp.zeros((_B, 2), jnp.float32) + b4_ref[0:1, :]
        for k in range(16):
            acc4 = acc4 + h3[:, k:k + 1] * w4_ref[k:k + 1, :]
        out = _softplus(acc4)
        ns_ref[0:1, 0:1] = jnp.mean(out[:, 0:1], keepdims=True)
        ns_ref[1:2, 0:1] = jnp.mean(out[:, 1:2], keepdims=True)
        m_ref[...] = jnp.full((_B, 1), _NEG, jnp.float32)
        s_acc_ref[...] = jnp.zeros((_B, 1), jnp.float32)
        mw_ref[...] = jnp.full((1, 1), _NEG, jnp.float32)
        sw_ref[...] = jnp.zeros((1, 1), jnp.float32)

    noise_scale = ns_ref[0:1, 0:1]
    correct_scale = ns_ref[1:2, 0:1]

    col = jax.lax.broadcasted_iota(jnp.int32, (1, _BLK), 1) + i * _BLK
    valid = col < _N

    ps = res_ref[...] + nz_ref[...] * (sig_ref[...] * noise_scale)
    a = _softplus(ps[0:1, :])
    slope = _softplus(ps[1:2, :])
    sg = _softplus(ps[2:3, :]) + 0.001
    inv_sg = 1.0 / sg
    ln_sg = jnp.log(sg)

    d = (t_ref[...] - (a + slope * s_ref[...])) * inv_sg
    log_lik = -0.5 * jnp.mean(d * d, axis=0, keepdims=True) - ln_sg - _C
    log_w = correct_scale * log_lik

    zw = jnp.where(valid, log_w, -jnp.inf)
    bmw = jnp.maximum(jnp.max(zw, keepdims=True), _NEG)
    bsw = jnp.sum(jnp.exp(zw - bmw), keepdims=True)
    mw_old = mw_ref[...]
    mw_new = jnp.maximum(mw_old, bmw)
    sw_ref[...] = sw_ref[...] * jnp.exp(mw_old - mw_new) + bsw * jnp.exp(bmw - mw_new)
    mw_ref[...] = mw_new

    ec = (e_ref[...] - a) * inv_sg
    z = jnp.where(valid, log_w + (-0.5 * ec * ec - ln_sg - _C), -jnp.inf)
    bm = jnp.maximum(jnp.max(z, axis=1, keepdims=True), _NEG)
    bs = jnp.sum(jnp.exp(z - bm), axis=1, keepdims=True)
    m_old = m_ref[...]
    m_new = jnp.maximum(m_old, bm)
    s_acc_ref[...] = s_acc_ref[...] * jnp.exp(m_old - m_new) + bs * jnp.exp(bm - m_new)
    m_ref[...] = m_new

    @pl.when(i == _NBLK - 1)
    def _fin():
        lse_b = m_ref[...] + jnp.log(s_acc_ref[...])
        lse_w = mw_ref[...] + jnp.log(sw_ref[...])
        out_ref[...] = -jnp.mean(lse_b, axis=0, keepdims=True) + lse_w


def _tc_loss(res_t, noise_t, t_obs, s_obs, eol, sigma, params, interpret=False):
    small = pl.BlockSpec((_B, 1), lambda i: (0, 0))
    big = pl.BlockSpec((3, _BLK), lambda i: (0, i))
    w_spec = lambda shp: pl.BlockSpec(shp, lambda i: (0, 0))
    (w1, b1, w2, b2, w3, b3, w4, b4) = params
    return pl.pallas_call(
        _tc_loss_body,
        grid=(_NBLK,),
        in_specs=[
            big, big, small, small, small,
            pl.BlockSpec((3, 1), lambda i: (0, 0)),
            w_spec((2, 32)), w_spec((1, 32)),
            w_spec((32, 32)), w_spec((1, 32)),
            w_spec((32, 16)), w_spec((1, 16)),
            w_spec((16, 2)), w_spec((1, 2)),
        ],
        out_specs=pl.BlockSpec((1, 1), lambda i: (0, 0)),
        out_shape=jax.ShapeDtypeStruct((1, 1), jnp.float32),
        scratch_shapes=[
            pltpu.VMEM((2, 1), jnp.float32),
            pltpu.VMEM((_B, 1), jnp.float32),
            pltpu.VMEM((_B, 1), jnp.float32),
            pltpu.VMEM((1, 1), jnp.float32),
            pltpu.VMEM((1, 1), jnp.float32),
        ],
        interpret=interpret,
    )(res_t, noise_t,
      t_obs.reshape(_B, 1), s_obs.reshape(_B, 1), eol.reshape(_B, 1),
      sigma.reshape(3, 1),
      w1, b1.reshape(1, 32), w2, b2.reshape(1, 32),
      w3, b3.reshape(1, 16), w4, b4.reshape(1, 2))


def kernel(t_obs, s_obs, eol, states, weights, onsets, sigma, W1, b1, W2, b2, W3, b3, W4, b4):
    n = weights.shape[0]
    key = jax.random.key(42)
    k_res, k_noise = jax.random.split(key)
    u = jax.random.uniform(k_res, (n,), dtype=jnp.float32)
    noise = jax.random.normal(k_noise, (n, 3), dtype=jnp.float32)

    cdf = jnp.cumsum(weights)
    idx = jnp.clip(jnp.searchsorted(cdf, u), 0, n - 1)
    res_t = jnp.take(states, idx, axis=0).T
    noise_t = noise.T

    loss = _tc_loss(res_t, noise_t, t_obs, s_obs, eol, sigma,
                    (W1, b1, W2, b2, W3, b3, W4, b4))
    return loss.reshape(())


# speedup vs baseline: 2.7757x; 2.7757x over previous
import functools

import jax
import jax.numpy as jnp
import numpy as np
from jax import lax
from jax.experimental import pallas as pl
from jax.experimental.pallas import tpu as pltpu
from jax.experimental.pallas import tpu_sc as plsc

_N = 1000000
_B = 16
_BLK = 16384
_NBLK = 62
_C = float(0.5 * np.log(2.0 * np.pi))
_NEG = -1e30

_NW = 32
_PER_W = 31744
_NPAD = _NW * _PER_W
_CH = 512
_NCHUNK = _PER_W // _CH
_NV = _CH // 16
_NROWS = _N // 16
_CTAB = 65536


def _softplus(x):
    return jnp.maximum(x, 0.0) + jnp.log1p(jnp.exp(-jnp.abs(x)))


def _sc_resample(u_pad, coarse_pad, cdf2d, states_flat):
    mesh = plsc.VectorSubcoreMesh(core_axis_name="c", subcore_axis_name="s")

    @functools.partial(
        pl.kernel,
        out_type=jax.ShapeDtypeStruct((3 * _NPAD,), jnp.float32),
        mesh=mesh,
        compiler_params=pltpu.CompilerParams(
            needs_layout_passes=False, use_tc_tiling_on_sc=False),
        scratch_types=[
            pltpu.VMEM((_CTAB,), jnp.float32),
            pltpu.VMEM((_CH,), jnp.float32),
            pltpu.VMEM((_CH,), jnp.int32),
            pltpu.VMEM((_CH, 16), jnp.float32),
            pltpu.VMEM((_CH,), jnp.int32),
            pltpu.VMEM((_CH,), jnp.int32),
            pltpu.VMEM((_CH,), jnp.int32),
            pltpu.VMEM((_CH,), jnp.float32),
            pltpu.VMEM((_CH,), jnp.float32),
            pltpu.VMEM((_CH,), jnp.float32),
            pltpu.SemaphoreType.DMA,
        ],
    )
    def k(u_hbm, coarse_hbm, cdf2d_hbm, st_hbm, out_hbm,
          coarse_v, u_v, bkt_v, rows_v, i0_v, i1_v, i2_v, p0_v, p1_v, p2_v,
          sem):
        wid = lax.axis_index("s") * 2 + lax.axis_index("c")
        pltpu.sync_copy(coarse_hbm, coarse_v)
        base = wid * _PER_W

        def chunk_body(ci, carry):
            start = base + ci * _CH
            pltpu.sync_copy(u_hbm.at[pl.ds(start, _CH)], u_v)

            def coarse_body(vi, c2):
                uu = u_v[pl.ds(vi * 16, 16)]
                lo = jnp.zeros((16,), jnp.int32)
                for sz in (32768, 16384, 8192, 4096, 2048, 1024, 512, 256,
                           128, 64, 32, 16, 8, 4, 2, 1):
                    cv = plsc.load_gather(coarse_v, [lo + (sz - 1)])
                    lo = jnp.where(cv < uu, lo + sz, lo)
                bkt_v[pl.ds(vi * 16, 16)] = jnp.minimum(lo, _NROWS - 1)
                return c2

            lax.fori_loop(0, _NV, coarse_body, 0)

            row_copies = [
                pltpu.async_copy(
                    cdf2d_hbm.at[bkt_v.at[pl.ds(j * 128, 128)]],
                    rows_v.at[pl.ds(j * 128, 128)], sem)
                for j in range(_CH // 128)
            ]
            for c in row_copies:
                c.wait()

            def fine_body(vi, c2):
                uu = u_v[pl.ds(vi * 16, 16)]
                bb = bkt_v[pl.ds(vi * 16, 16)]
                rowid = lax.iota(jnp.int32, 16) + vi * 16
                lo = jnp.zeros((16,), jnp.int32)
                for sz in (8, 4, 2, 1):
                    rv = plsc.load_gather(rows_v, [rowid, lo + (sz - 1)])
                    lo = jnp.where(rv < uu, lo + sz, lo)
                i3 = jnp.minimum(bb * 16 + lo, _N - 1) * 3
                sl = pl.ds(vi * 16, 16)
                i0_v[sl] = i3
                i1_v[sl] = i3 + 1
                i2_v[sl] = i3 + 2
                return c2

            lax.fori_loop(0, _NV, fine_body, 0)

            plane_copies = []
            for iv, pv in ((i0_v, p0_v), (i1_v, p1_v), (i2_v, p2_v)):
                for j in range(_CH // 128):
                    plane_copies.append(pltpu.async_copy(
                        st_hbm.at[iv.at[pl.ds(j * 128, 128)]],
                        pv.at[pl.ds(j * 128, 128)], sem))
            for c in plane_copies:
                c.wait()

            pltpu.sync_copy(p0_v, out_hbm.at[pl.ds(start, _CH)])
            pltpu.sync_copy(p1_v, out_hbm.at[pl.ds(_NPAD + start, _CH)])
            pltpu.sync_copy(p2_v, out_hbm.at[pl.ds(2 * _NPAD + start, _CH)])
            return carry

        lax.fori_loop(0, _NCHUNK, chunk_body, 0)

    return k(u_pad, coarse_pad, cdf2d, states_flat)


def _tc_loss_body(res_ref, nz_ref, t_ref, s_ref, e_ref, sig_ref, nscs_ref,
                  w1_ref, b1_ref, w2_ref, b2_ref, w3_ref, b3_ref, w4_ref, b4_ref,
                  out_ref, ns_ref, m_ref, s_acc_ref, mw_ref, sw_ref):
    i = pl.program_id(0)

    @pl.when(i == 0)
    def _init():
        ns_ref[0:1, 0:1] = nscs_ref[0:1, 0:1]
        ns_ref[1:2, 0:1] = nscs_ref[1:2, 0:1]
        m_ref[...] = jnp.full((_B, 1), _NEG, jnp.float32)
        s_acc_ref[...] = jnp.zeros((_B, 1), jnp.float32)
        mw_ref[...] = jnp.full((1, 1), _NEG, jnp.float32)
        sw_ref[...] = jnp.zeros((1, 1), jnp.float32)

    noise_scale = ns_ref[0:1, 0:1]
    correct_scale = ns_ref[1:2, 0:1]

    col = jax.lax.broadcasted_iota(jnp.int32, (1, _BLK), 1) + i * _BLK
    valid = col < _N

    ps = res_ref[...] + nz_ref[...] * (sig_ref[...] * noise_scale)
    a = _softplus(ps[0:1, :])
    slope = _softplus(ps[1:2, :])
    sg = _softplus(ps[2:3, :]) + 0.001
    inv_sg = 1.0 / sg
    ln_sg = jnp.log(sg)

    d = (t_ref[...] - (a + slope * s_ref[...])) * inv_sg
    log_lik = -0.5 * jnp.mean(d * d, axis=0, keepdims=True) - ln_sg - _C
    log_w = correct_scale * log_lik

    zw = jnp.where(valid, log_w, -jnp.inf)
    bmw = jnp.maximum(jnp.max(zw, keepdims=True), _NEG)
    bsw = jnp.sum(jnp.exp(zw - bmw), keepdims=True)
    mw_old = mw_ref[...]
    mw_new = jnp.maximum(mw_old, bmw)
    sw_ref[...] = sw_ref[...] * jnp.exp(mw_old - mw_new) + bsw * jnp.exp(bmw - mw_new)
    mw_ref[...] = mw_new

    ec = (e_ref[...] - a) * inv_sg
    z = jnp.where(valid, log_w + (-0.5 * ec * ec - ln_sg - _C), -jnp.inf)
    bm = jnp.maximum(jnp.max(z, axis=1, keepdims=True), _NEG)
    bs = jnp.sum(jnp.exp(z - bm), axis=1, keepdims=True)
    m_old = m_ref[...]
    m_new = jnp.maximum(m_old, bm)
    s_acc_ref[...] = s_acc_ref[...] * jnp.exp(m_old - m_new) + bs * jnp.exp(bm - m_new)
    m_ref[...] = m_new

    @pl.when(i == _NBLK - 1)
    def _fin():
        lse_b = m_ref[...] + jnp.log(s_acc_ref[...])
        lse_w = mw_ref[...] + jnp.log(sw_ref[...])
        out_ref[...] = -jnp.mean(lse_b, axis=0, keepdims=True) + lse_w


def _tc_loss(res_t, noise_t, t_obs, s_obs, eol, sigma, nscs, params, interpret=False):
    small = pl.BlockSpec((_B, 1), lambda i: (0, 0))
    big = pl.BlockSpec((3, _BLK), lambda i: (0, i))
    w_spec = lambda shp: pl.BlockSpec(shp, lambda i: (0, 0))
    (w1, b1, w2, b2, w3, b3, w4, b4) = params
    return pl.pallas_call(
        _tc_loss_body,
        grid=(_NBLK,),
        in_specs=[
            big, big, small, small, small,
            pl.BlockSpec((3, 1), lambda i: (0, 0)),
            pl.BlockSpec((2, 1), lambda i: (0, 0)),
            w_spec((2, 32)), w_spec((1, 32)),
            w_spec((32, 32)), w_spec((1, 32)),
            w_spec((32, 16)), w_spec((1, 16)),
            w_spec((16, 2)), w_spec((1, 2)),
        ],
        out_specs=pl.BlockSpec((1, 1), lambda i: (0, 0)),
        out_shape=jax.ShapeDtypeStruct((1, 1), jnp.float32),
        scratch_shapes=[
            pltpu.VMEM((2, 1), jnp.float32),
            pltpu.VMEM((_B, 1), jnp.float32),
            pltpu.VMEM((_B, 1), jnp.float32),
            pltpu.VMEM((1, 1), jnp.float32),
            pltpu.VMEM((1, 1), jnp.float32),
        ],
        interpret=interpret,
    )(res_t, noise_t,
      t_obs.reshape(_B, 1), s_obs.reshape(_B, 1), eol.reshape(_B, 1),
      sigma.reshape(3, 1), nscs,
      w1, b1.reshape(1, 32), w2, b2.reshape(1, 32),
      w3, b3.reshape(1, 16), w4, b4.reshape(1, 2))


def kernel(t_obs, s_obs, eol, states, weights, onsets, sigma, W1, b1, W2, b2, W3, b3, W4, b4):
    n = weights.shape[0]
    key = jax.random.key(42)
    k_res, k_noise = jax.random.split(key)
    u = jax.random.uniform(k_res, (n,), dtype=jnp.float32)
    noise = jax.random.normal(k_noise, (n, 3), dtype=jnp.float32)

    cdf = jnp.cumsum(weights)
    u_pad = jnp.zeros((_NPAD,), jnp.float32).at[:n].set(u)
    coarse_pad = jnp.full((_CTAB,), 2.0, jnp.float32).at[:_NROWS].set(cdf[15::16])
    cdf2d = cdf.reshape(_NROWS, 16)
    states_flat = states.reshape(3 * n)

    res_t = _sc_resample(u_pad, coarse_pad, cdf2d, states_flat).reshape(3, _NPAD)
    noise_t = noise.T

    x = jnp.stack([t_obs, s_obs], axis=-1)
    h = jax.nn.relu(x @ W1 + b1)
    h = jax.nn.relu(h @ W2 + b2)
    h = jax.nn.relu(h @ W3 + b3)
    out = jax.nn.softplus(h @ W4 + b4)
    nscs = jnp.stack([out[..., :-1].mean(), out[..., -1].mean()]).reshape(2, 1)

    loss = _tc_loss(res_t, noise_t, t_obs, s_obs, eol, sigma, nscs,
                    (W1, b1, W2, b2, W3, b3, W4, b4))
    return loss.reshape(())
